# Initial kernel scaffold; baseline (speedup 1.0000x reference)
import jax
import jax.numpy as jnp
from jax.experimental import pallas as pl


def kernel(features, W_conv, b_conv, W_cls, b_cls, W_bbox, b_bbox):
    raise NotImplementedError("write your pallas kernel here")


# speedup vs baseline: 2.3578x; 2.3578x over previous
import functools

import jax
import jax.numpy as jnp
from jax.experimental import pallas as pl


def _rpn_kernel(x_ref, top_ref, bot_ref, wt_ref, wh_ref, bc_ref, bh_ref,
                cls_ref, bbox_ref, *, C, Rb, W, RS, k):
    i = pl.program_id(1)
    n_i = pl.num_programs(1)
    top_row = jnp.where(i > 0, top_ref[0, :, 7:8, :], 0.0)
    bot_row = jnp.where(i < n_i - 1, bot_ref[0, :, 0:1, :], 0.0)

    n = RS * W
    for r0 in range(0, Rb, RS):
        if r0 == 0:
            xe = jnp.concatenate([top_row, x_ref[0, :, 0:RS + 1, :]], axis=1)
        elif r0 + RS == Rb:
            xe = jnp.concatenate([x_ref[0, :, r0 - 1:Rb, :], bot_row], axis=1)
        else:
            xe = x_ref[0, :, r0 - 1:r0 + RS + 1, :]
        xe = xe.astype(jnp.bfloat16)

        zcol = jnp.zeros((C, RS + 2, 1), dtype=jnp.bfloat16)
        x_dx = (
            jnp.concatenate([zcol, xe[:, :, :-1]], axis=2),
            xe,
            jnp.concatenate([xe[:, :, 1:], zcol], axis=2),
        )
        x_flat = tuple(v.reshape(C, (RS + 2) * W) for v in x_dx)

        acc = jnp.zeros((C, n), dtype=jnp.float32)
        for dy in range(3):
            lo = dy * W
            for dx in range(3):
                acc += jnp.dot(wt_ref[3 * dy + dx],
                               x_flat[dx][:, lo:lo + n],
                               preferred_element_type=jnp.float32)
        acc += bc_ref[:]
        t = jax.nn.relu(acc).astype(jnp.bfloat16)

        u = jnp.dot(wh_ref[:], t, preferred_element_type=jnp.float32)
        u = (u + bh_ref[:]).reshape(48, RS, W)
        cls_ref[0, :, r0:r0 + RS, :] = jax.nn.sigmoid(u[0:k])
        bbox_ref[0, :, r0:r0 + RS, :] = u[k:5 * k]


def kernel(features, W_conv, b_conv, W_cls, b_cls, W_bbox, b_bbox):
    B, C, H, W = features.shape
    k = W_cls.shape[0]
    NH = 2
    Rb = H // NH
    RS = 16

    wt = jnp.transpose(W_conv, (2, 3, 0, 1)).reshape(9, C, C)
    wt = wt.astype(jnp.bfloat16)
    wh = jnp.concatenate([W_cls[:, :, 0, 0], W_bbox[:, :, 0, 0]], axis=0)
    wh = jnp.pad(wh, ((0, 48 - 5 * k), (0, 0))).astype(jnp.bfloat16)
    bc = b_conv.reshape(C, 1)
    bh = jnp.pad(jnp.concatenate([b_cls, b_bbox]), (0, 48 - 5 * k))
    bh = bh.reshape(48, 1)

    nh8 = H // 8
    in_specs = [
        pl.BlockSpec((1, C, Rb, W), lambda b, i: (b, 0, i, 0)),
        pl.BlockSpec((1, C, 8, W),
                     lambda b, i: (b, 0, jnp.maximum(i * (Rb // 8) - 1, 0), 0)),
        pl.BlockSpec((1, C, 8, W),
                     lambda b, i: (b, 0,
                                   jnp.minimum((i + 1) * (Rb // 8), nh8 - 1),
                                   0)),
        pl.BlockSpec((9, C, C), lambda b, i: (0, 0, 0)),
        pl.BlockSpec((48, C), lambda b, i: (0, 0)),
        pl.BlockSpec((C, 1), lambda b, i: (0, 0)),
        pl.BlockSpec((48, 1), lambda b, i: (0, 0)),
    ]
    out_specs = [
        pl.BlockSpec((1, k, Rb, W), lambda b, i: (b, 0, i, 0)),
        pl.BlockSpec((1, 4 * k, Rb, W), lambda b, i: (b, 0, i, 0)),
    ]
    out_shape = [
        jax.ShapeDtypeStruct((B, k, H, W), jnp.float32),
        jax.ShapeDtypeStruct((B, 4 * k, H, W), jnp.float32),
    ]
    cls_score, bbox_pred = pl.pallas_call(
        functools.partial(_rpn_kernel, C=C, Rb=Rb, W=W, RS=RS, k=k),
        grid=(B, NH),
        in_specs=in_specs,
        out_specs=out_specs,
        out_shape=out_shape,
    )(features, features, features, wt, wh, bc, bh)
    return (cls_score, bbox_pred)
